# baseline (speedup 1.0000x reference)
import functools

import jax
import jax.numpy as jnp
from jax import lax
from jax.experimental import pallas as pl
from jax.experimental.pallas import tpu as pltpu
from jax.experimental.pallas import tpu_sc as plsc

NS = 16
L = 16


def _zmat_body(C, D, x_ref, w_ref, wp_ref, b_ref, bp_ref, z_ref, c_ref):
    wv = jnp.dot(w_ref[...], wp_ref[...], preferred_element_type=jnp.float32)
    xm = x_ref[:, 0, :]
    for c in range(1, C):
        xm = xm + x_ref[:, c, :]
    xm = xm * (1.0 / C)
    z_ref[...] = jnp.dot(xm, wv, preferred_element_type=jnp.float32)
    c_ref[...] = jnp.dot(b_ref[...], wv, preferred_element_type=jnp.float32) + bp_ref[...]


def _sc_body(src_hbm, ew_hbm, dst_hbm, z_hbm, i0_hbm, i1_hbm, c16_hbm, out_hbm,
             src_f, ew_f, msg_f, dst_f, z_v, buf_v, i0_v, i1_v, o_v, c_v,
             shared_a, sem):
    sid = lax.axis_index("s")
    ept = src_f.shape[0]
    slc = buf_v.shape[0]
    spt = i0_v.shape[0]
    ebase = sid * ept

    cps = [pltpu.async_copy(src_hbm.at[pl.ds(ebase, ept)], src_f, sem),
           pltpu.async_copy(ew_hbm.at[pl.ds(ebase, ept)], ew_f, sem),
           pltpu.async_copy(dst_hbm.at[pl.ds(ebase, ept)], dst_f, sem),
           pltpu.async_copy(z_hbm, z_v, sem)]

    def zero_loop(i, carry):
        buf_v[pl.ds(i * L, L)] = jnp.zeros((L,), jnp.float32)
        return carry

    lax.fori_loop(0, slc // L, zero_loop, 0)
    for cp in cps:
        cp.wait()
    pltpu.sync_copy(buf_v, shared_a.at[pl.ds(sid * slc, slc)])
    plsc.subcore_barrier()

    def msg_loop(i, carry):
        s16 = src_f[pl.ds(i * L, L)]
        w16 = ew_f[pl.ds(i * L, L)]
        msg_f[pl.ds(i * L, L)] = plsc.load_gather(z_v, [s16]) * w16
        return carry

    lax.fori_loop(0, ept // L, msg_loop, 0)

    pltpu.sync_copy(msg_f, shared_a.at[dst_f], add=True)
    plsc.subcore_barrier()

    cps = [pltpu.async_copy(i0_hbm.at[pl.ds(sid * spt, spt)], i0_v, sem),
           pltpu.async_copy(i1_hbm.at[pl.ds(sid * spt, spt)], i1_v, sem),
           pltpu.async_copy(c16_hbm, c_v, sem)]
    pltpu.sync_copy(shared_a, z_v)
    for cp in cps:
        cp.wait()
    cv = c_v[...]

    def g_loop(k, carry):
        x0 = i0_v[pl.ds(k * L, L)]
        x1 = i1_v[pl.ds(k * L, L)]
        g = plsc.load_gather(z_v, [x0]) + plsc.load_gather(z_v, [x1])
        o_v[pl.ds(k * L, L)] = g * 0.5 + cv
        return carry

    lax.fori_loop(0, spt // L, g_loop, 0)
    pltpu.sync_copy(o_v, out_hbm.at[pl.ds(sid * spt, spt)])


def kernel(x, edge_index, edge_weight, subG_node, W, b, Wp, bp):
    N, C, D = x.shape
    E = edge_index.shape[1]
    S, K = subG_node.shape
    ept = E // NS
    spt = S // NS
    assert K == 2 and S % (NS * L) == 0 and E == ept * NS and ept % L == 0

    z2, c2 = pl.pallas_call(
        functools.partial(_zmat_body, C, D),
        out_shape=(jax.ShapeDtypeStruct((N, 1), jnp.float32),
                   jax.ShapeDtypeStruct((1, 1), jnp.float32)),
    )(x, W, Wp, b.reshape(1, D), bp.reshape(1, 1))
    c16 = jnp.broadcast_to(c2.reshape(1), (L,))

    npad = -(-N // (NS * L)) * (NS * L)
    slc = npad // NS
    z = jnp.concatenate([z2.reshape(N), jnp.zeros((npad - N,), jnp.float32)])

    src = edge_index[0]
    dst = edge_index[1]
    i0 = subG_node[:, 0]
    i1 = subG_node[:, 1]

    mesh = plsc.VectorSubcoreMesh(core_axis_name="c", subcore_axis_name="s",
                                  num_cores=1, num_subcores=NS)
    sc_params = pltpu.CompilerParams(needs_layout_passes=False)

    sc = pl.kernel(
        _sc_body,
        out_type=jax.ShapeDtypeStruct((S,), jnp.float32),
        mesh=mesh,
        compiler_params=sc_params,
        scratch_types=[
            pltpu.VMEM((ept,), jnp.int32),
            pltpu.VMEM((ept,), jnp.float32),
            pltpu.VMEM((ept,), jnp.float32),
            pltpu.VMEM((ept,), jnp.int32),
            pltpu.VMEM((npad,), jnp.float32),
            pltpu.VMEM((slc,), jnp.float32),
            pltpu.VMEM((spt,), jnp.int32),
            pltpu.VMEM((spt,), jnp.int32),
            pltpu.VMEM((spt,), jnp.float32),
            pltpu.VMEM((L,), jnp.float32),
            pltpu.VMEM_SHARED((npad,), jnp.float32),
            pltpu.SemaphoreType.DMA,
        ],
    )
    out = sc(src, edge_weight, dst, z, i0, i1, c16)
    return out.reshape(S, 1)


# speedup vs baseline: 1.0212x; 1.0212x over previous
import functools

import jax
import jax.numpy as jnp
from jax import lax
from jax.experimental import pallas as pl
from jax.experimental.pallas import tpu as pltpu
from jax.experimental.pallas import tpu_sc as plsc

NS = 16
L = 16
NCK = 5


def _zmat_body(C, x_ref, w_ref, wp_ref, b_ref, bp_ref, z_ref, c_ref):
    n, _, d = x_ref.shape
    wv = jnp.dot(w_ref[...], wp_ref[...], preferred_element_type=jnp.float32)
    x2 = x_ref[...].reshape(n * C, d)
    z_ref[...] = jnp.dot(x2, wv, preferred_element_type=jnp.float32)
    c_ref[...] = jnp.dot(b_ref[...], wv,
                         preferred_element_type=jnp.float32) + bp_ref[...]


def _sc_body(C, src_hbm, ew_hbm, dst_hbm, z_hbm, i0_hbm, i1_hbm, c16_hbm,
             out_hbm, src_f, ew_f, msg_f, dst_c, z_v, a_v, buf_v, i0_v, i1_v,
             o_v, c_v, shared_a, sem, ssem):
    sid = lax.axis_index("s")
    ept = src_f.shape[0]
    slc = buf_v.shape[0]
    spt = i0_v.shape[0]
    csz = ept // NCK
    ebase = sid * ept

    cps = [pltpu.async_copy(src_hbm.at[pl.ds(ebase, ept)], src_f, sem),
           pltpu.async_copy(ew_hbm.at[pl.ds(ebase, ept)], ew_f, sem)]
    cps += [pltpu.async_copy(dst_hbm.at[pl.ds(ebase + k * csz, csz)],
                             dst_c[k], sem) for k in range(NCK)]
    cps.append(pltpu.async_copy(z_hbm, z_v, sem))

    @plsc.parallel_loop(0, slc, L, unroll=4)
    def _(i):
        buf_v[pl.ds(i, L)] = jnp.zeros((L,), jnp.float32)

    for cp in cps:
        cp.wait()
    pltpu.sync_copy(buf_v, shared_a.at[pl.ds(sid * slc, slc)])
    plsc.subcore_barrier()

    for k in range(NCK):
        @plsc.parallel_loop(k * csz, (k + 1) * csz, L, unroll=8)
        def _(i):
            sC = src_f[pl.ds(i, L)] * C
            w16 = ew_f[pl.ds(i, L)] * (1.0 / C)
            g = plsc.load_gather(z_v, [sC])
            for c in range(1, C):
                g = g + plsc.load_gather(z_v, [sC + c])
            msg_f[pl.ds(i, L)] = g * w16

        pltpu.async_copy(msg_f.at[pl.ds(k * csz, csz)],
                         shared_a.at[dst_c[k]], ssem, add=True)

    for k in range(NCK):
        pltpu.make_async_copy(msg_f.at[pl.ds(k * csz, csz)],
                              shared_a.at[dst_c[k]], ssem).wait()
    plsc.subcore_barrier()

    cps = [pltpu.async_copy(i0_hbm.at[pl.ds(sid * spt, spt)], i0_v, sem),
           pltpu.async_copy(i1_hbm.at[pl.ds(sid * spt, spt)], i1_v, sem),
           pltpu.async_copy(c16_hbm, c_v, sem)]
    pltpu.sync_copy(shared_a, a_v)
    for cp in cps:
        cp.wait()
    cv = c_v[...]

    @plsc.parallel_loop(0, spt, L, unroll=4)
    def _(k):
        x0 = i0_v[pl.ds(k, L)]
        x1 = i1_v[pl.ds(k, L)]
        g = plsc.load_gather(a_v, [x0]) + plsc.load_gather(a_v, [x1])
        o_v[pl.ds(k, L)] = g * 0.5 + cv

    pltpu.sync_copy(o_v, out_hbm.at[pl.ds(sid * spt, spt)])


def kernel(x, edge_index, edge_weight, subG_node, W, b, Wp, bp):
    N, C, D = x.shape
    E = edge_index.shape[1]
    S, K = subG_node.shape
    ept = E // NS
    spt = S // NS
    assert (K == 2 and S % (NS * L) == 0 and E == ept * NS
            and ept % (NCK * L) == 0 and (ept // NCK) % 8 == 0)

    z2, c2 = pl.pallas_call(
        functools.partial(_zmat_body, C),
        out_shape=(jax.ShapeDtypeStruct((N * C, 1), jnp.float32),
                   jax.ShapeDtypeStruct((1, 1), jnp.float32)),
    )(x, W, Wp, b.reshape(1, D), bp.reshape(1, 1))
    c16 = jnp.broadcast_to(c2.reshape(1), (L,))

    npad = -(-N // (NS * L)) * (NS * L)
    zpad = -(-(N * C) // (NS * L)) * (NS * L)
    slc = npad // NS
    z = jnp.concatenate([z2.reshape(N * C),
                         jnp.zeros((zpad - N * C,), jnp.float32)])

    src = edge_index[0]
    dst = edge_index[1]
    i0 = subG_node[:, 0]
    i1 = subG_node[:, 1]

    mesh = plsc.VectorSubcoreMesh(core_axis_name="c", subcore_axis_name="s",
                                  num_cores=1, num_subcores=NS)
    sc_params = pltpu.CompilerParams(needs_layout_passes=False)

    sc = pl.kernel(
        functools.partial(_sc_body, C),
        out_type=jax.ShapeDtypeStruct((S,), jnp.float32),
        mesh=mesh,
        compiler_params=sc_params,
        scratch_types=[
            pltpu.VMEM((ept,), jnp.int32),
            pltpu.VMEM((ept,), jnp.float32),
            pltpu.VMEM((ept,), jnp.float32),
            [pltpu.VMEM((ept // NCK,), jnp.int32) for _ in range(NCK)],
            pltpu.VMEM((zpad,), jnp.float32),
            pltpu.VMEM((npad,), jnp.float32),
            pltpu.VMEM((slc,), jnp.float32),
            pltpu.VMEM((spt,), jnp.int32),
            pltpu.VMEM((spt,), jnp.int32),
            pltpu.VMEM((spt,), jnp.float32),
            pltpu.VMEM((L,), jnp.float32),
            pltpu.VMEM_SHARED((npad,), jnp.float32),
            pltpu.SemaphoreType.DMA,
            pltpu.SemaphoreType.DMA,
        ],
    )
    out = sc(src, edge_weight, dst, z, i0, i1, c16)
    return out.reshape(S, 1)


# speedup vs baseline: 1.2032x; 1.1783x over previous
import functools

import jax
import jax.numpy as jnp
from jax import lax
from jax.experimental import pallas as pl
from jax.experimental.pallas import tpu as pltpu
from jax.experimental.pallas import tpu_sc as plsc

NS = 16
L = 16
NCK = 5


def _zmat_body(C, x_ref, w_ref, wp_ref, b_ref, bp_ref, z_ref, c_ref):
    g = pl.program_id(0)
    wv = jnp.dot(w_ref[...], wp_ref[...],
                 preferred_element_type=jnp.float32) * (1.0 / C)
    xs = x_ref[:, 0, :]
    for c in range(1, C):
        xs = xs + x_ref[:, c, :]
    z_ref[...] = jnp.dot(xs, wv, preferred_element_type=jnp.float32)

    @pl.when(g == 0)
    def _():
        c1 = jnp.dot(b_ref[...], wv,
                     preferred_element_type=jnp.float32) * C + bp_ref[...]
        c_ref[...] = jnp.dot(c1, jnp.ones((1, L), jnp.float32),
                             preferred_element_type=jnp.float32)


def _sc_body(C, ei_hbm, ew_hbm, z_hbm, sg_hbm, c16_hbm, out_hbm,
             src_f, ew_f, msg_f, dst_c, z_v, a_v, buf_v, sg_v, o_v, c_v,
             shared_a, sem, ssem):
    sid = lax.axis_index("s")
    ept = src_f.shape[0]
    slc = buf_v.shape[0]
    spt = o_v.shape[0]
    csz = ept // NCK
    ebase = sid * ept

    zero16 = jnp.zeros((L,), jnp.int32)
    cps = [pltpu.async_copy(ei_hbm.at[0, pl.ds(ebase, ept)], src_f, sem),
           pltpu.async_copy(ew_hbm.at[pl.ds(ebase, ept)], ew_f, sem)]
    cps += [pltpu.async_copy(ei_hbm.at[1, pl.ds(ebase + k * csz, csz)],
                             dst_c[k], sem) for k in range(NCK)]
    cps.append(pltpu.async_copy(z_hbm, z_v.at[pl.ds(0, z_hbm.shape[0])], sem))
    cps.append(pltpu.async_copy(c16_hbm.at[0], c_v, sem))

    @plsc.parallel_loop(0, slc, L, unroll=4)
    def _(i):
        buf_v[pl.ds(i, L)] = jnp.zeros((L,), jnp.float32)

    for cp in cps:
        cp.wait()
    pltpu.sync_copy(buf_v, shared_a.at[pl.ds(sid * slc, slc)])
    plsc.subcore_barrier()

    for k in range(NCK):
        @plsc.parallel_loop(k * csz, (k + 1) * csz, L, unroll=8)
        def _(i):
            s16 = src_f[pl.ds(i, L)]
            w16 = ew_f[pl.ds(i, L)]
            msg_f[pl.ds(i, L)] = plsc.load_gather(z_v, [s16]) * w16

        pltpu.async_copy(msg_f.at[pl.ds(k * csz, csz)],
                         shared_a.at[dst_c[k]], ssem, add=True)

    for k in range(NCK):
        pltpu.make_async_copy(msg_f.at[pl.ds(k * csz, csz)],
                              shared_a.at[dst_c[k]], ssem).wait()
    plsc.subcore_barrier()

    cp = pltpu.async_copy(sg_hbm.at[pl.ds(sid * spt, spt), :], sg_v, sem)
    pltpu.sync_copy(shared_a, a_v)
    cp.wait()
    cv = c_v[...]
    iota = jax.lax.iota(jnp.int32, L)

    @plsc.parallel_loop(0, spt, L, unroll=4)
    def _(k):
        row = iota + k
        x0 = plsc.load_gather(sg_v, [row, zero16])
        x1 = plsc.load_gather(sg_v, [row, zero16 + 1])
        g = plsc.load_gather(a_v, [x0]) + plsc.load_gather(a_v, [x1])
        o_v[pl.ds(k, L)] = g * 0.5 + cv

    pltpu.sync_copy(o_v, out_hbm.at[pl.ds(sid * spt, spt)])


def kernel(x, edge_index, edge_weight, subG_node, W, b, Wp, bp):
    N, C, D = x.shape
    E = edge_index.shape[1]
    S, K = subG_node.shape
    ept = E // NS
    spt = S // NS
    assert (K == 2 and S % (NS * L) == 0 and E == ept * NS
            and ept % (NCK * L) == 0 and (ept // NCK) % 8 == 0)

    GB = 5
    bn = N // GB
    assert N == GB * bn and bn % 8 == 0
    z2, c16 = pl.pallas_call(
        functools.partial(_zmat_body, C),
        grid=(GB,),
        in_specs=[pl.BlockSpec((bn, C, D), lambda g: (g, 0, 0)),
                  pl.BlockSpec((D, D), lambda g: (0, 0)),
                  pl.BlockSpec((D, 1), lambda g: (0, 0)),
                  pl.BlockSpec((1, D), lambda g: (0, 0)),
                  pl.BlockSpec((1, 1), lambda g: (0, 0))],
        out_specs=(pl.BlockSpec((bn, 1), lambda g: (g, 0)),
                   pl.BlockSpec((1, L), lambda g: (0, 0))),
        out_shape=(jax.ShapeDtypeStruct((N, 1), jnp.float32),
                   jax.ShapeDtypeStruct((1, L), jnp.float32)),
    )(x, W, Wp, b.reshape(1, D), bp.reshape(1, 1))

    npad = -(-N // (NS * L)) * (NS * L)
    slc = npad // NS

    mesh = plsc.VectorSubcoreMesh(core_axis_name="c", subcore_axis_name="s",
                                  num_cores=1, num_subcores=NS)
    sc_params = pltpu.CompilerParams(needs_layout_passes=False,
                                     use_tc_tiling_on_sc=False)

    sc = pl.kernel(
        functools.partial(_sc_body, C),
        out_type=jax.ShapeDtypeStruct((S,), jnp.float32),
        mesh=mesh,
        compiler_params=sc_params,
        scratch_types=[
            pltpu.VMEM((ept,), jnp.int32),
            pltpu.VMEM((ept,), jnp.float32),
            pltpu.VMEM((ept,), jnp.float32),
            [pltpu.VMEM((ept // NCK,), jnp.int32) for _ in range(NCK)],
            pltpu.VMEM((npad,), jnp.float32),
            pltpu.VMEM((npad,), jnp.float32),
            pltpu.VMEM((slc,), jnp.float32),
            pltpu.VMEM((spt, K), jnp.int32),
            pltpu.VMEM((spt,), jnp.float32),
            pltpu.VMEM((L,), jnp.float32),
            pltpu.VMEM_SHARED((npad,), jnp.float32),
            pltpu.SemaphoreType.DMA,
            pltpu.SemaphoreType.DMA,
        ],
    )
    out = sc(edge_index, edge_weight, z2.reshape(N), subG_node, c16)
    return out.reshape(S, 1)


# speedup vs baseline: 1.3147x; 1.0927x over previous
import functools

import jax
import jax.numpy as jnp
from jax import lax
from jax.experimental import pallas as pl
from jax.experimental.pallas import tpu as pltpu
from jax.experimental.pallas import tpu_sc as plsc

NS = 16
L = 16
NCK = 5


def _zmat_body(C, x_ref, w_ref, wp_ref, b_ref, bp_ref, z_ref, c_ref):
    g = pl.program_id(0)
    wv = jnp.dot(w_ref[...], wp_ref[...],
                 preferred_element_type=jnp.float32) * (1.0 / C)
    xs = x_ref[:, 0, :]
    for c in range(1, C):
        xs = xs + x_ref[:, c, :]
    z_ref[...] = jnp.dot(xs, wv, preferred_element_type=jnp.float32)

    @pl.when(g == 0)
    def _():
        c1 = jnp.dot(b_ref[...], wv,
                     preferred_element_type=jnp.float32) * C + bp_ref[...]
        c_ref[...] = jnp.dot(c1, jnp.ones((1, L), jnp.float32),
                             preferred_element_type=jnp.float32)


def _sc_body(C, ei_hbm, ew_hbm, z_hbm, i0_hbm, i1_hbm, c16_hbm, out_hbm,
             src_f, ew_f, msg_f, dst_c, z_v, a_v, buf_v, i0_v, i1_v, o_v, c_v,
             shared_a, sem, ssem):
    sid = lax.axis_index("s")
    ept = src_f.shape[0]
    slc = buf_v.shape[0]
    spt = o_v.shape[0]
    csz = ept // NCK
    ebase = sid * ept

    zero16 = jnp.zeros((L,), jnp.int32)
    cps = [pltpu.async_copy(ei_hbm.at[0, pl.ds(ebase, ept)], src_f, sem),
           pltpu.async_copy(ew_hbm.at[pl.ds(ebase, ept)], ew_f, sem)]
    cps += [pltpu.async_copy(ei_hbm.at[1, pl.ds(ebase + k * csz, csz)],
                             dst_c[k], sem) for k in range(NCK)]
    cps.append(pltpu.async_copy(z_hbm, z_v.at[pl.ds(0, z_hbm.shape[0])], sem))
    cps.append(pltpu.async_copy(c16_hbm.at[0], c_v, sem))

    @plsc.parallel_loop(0, slc, L, unroll=4)
    def _(i):
        buf_v[pl.ds(i, L)] = jnp.zeros((L,), jnp.float32)

    for cp in cps:
        cp.wait()
    pltpu.sync_copy(buf_v, shared_a.at[pl.ds(sid * slc, slc)])
    plsc.subcore_barrier()

    for k in range(NCK):
        @plsc.parallel_loop(k * csz, (k + 1) * csz, L, unroll=8)
        def _(i):
            s16 = src_f[pl.ds(i, L)]
            w16 = ew_f[pl.ds(i, L)]
            msg_f[pl.ds(i, L)] = plsc.load_gather(z_v, [s16]) * w16

        pltpu.async_copy(msg_f.at[pl.ds(k * csz, csz)],
                         shared_a.at[dst_c[k]], ssem, add=True)

    for k in range(NCK):
        pltpu.make_async_copy(msg_f.at[pl.ds(k * csz, csz)],
                              shared_a.at[dst_c[k]], ssem).wait()
    plsc.subcore_barrier()

    cps = [pltpu.async_copy(i0_hbm.at[pl.ds(sid * spt, spt)], i0_v, sem),
           pltpu.async_copy(i1_hbm.at[pl.ds(sid * spt, spt)], i1_v, sem)]
    pltpu.sync_copy(shared_a, a_v)
    for cp in cps:
        cp.wait()
    cv = c_v[...]

    @plsc.parallel_loop(0, spt, L, unroll=4)
    def _(k):
        x0 = i0_v[pl.ds(k, L)]
        x1 = i1_v[pl.ds(k, L)]
        g = plsc.load_gather(a_v, [x0]) + plsc.load_gather(a_v, [x1])
        o_v[pl.ds(k, L)] = g * 0.5 + cv

    pltpu.sync_copy(o_v, out_hbm.at[pl.ds(sid * spt, spt)])


def kernel(x, edge_index, edge_weight, subG_node, W, b, Wp, bp):
    N, C, D = x.shape
    E = edge_index.shape[1]
    S, K = subG_node.shape
    ept = E // NS
    spt = S // NS
    assert (K == 2 and S % (NS * L) == 0 and E == ept * NS
            and ept % (NCK * L) == 0 and (ept // NCK) % 8 == 0)

    GB = 10
    bn = N // GB
    assert N == GB * bn and bn % 8 == 0
    z2, c16 = pl.pallas_call(
        functools.partial(_zmat_body, C),
        grid=(GB,),
        in_specs=[pl.BlockSpec((bn, C, D), lambda g: (g, 0, 0)),
                  pl.BlockSpec((D, D), lambda g: (0, 0)),
                  pl.BlockSpec((D, 1), lambda g: (0, 0)),
                  pl.BlockSpec((1, D), lambda g: (0, 0)),
                  pl.BlockSpec((1, 1), lambda g: (0, 0))],
        out_specs=(pl.BlockSpec((bn, 1), lambda g: (g, 0)),
                   pl.BlockSpec((1, L), lambda g: (0, 0))),
        out_shape=(jax.ShapeDtypeStruct((N, 1), jnp.float32),
                   jax.ShapeDtypeStruct((1, L), jnp.float32)),
    )(x, W, Wp, b.reshape(1, D), bp.reshape(1, 1))

    npad = -(-N // (NS * L)) * (NS * L)
    slc = npad // NS

    mesh = plsc.VectorSubcoreMesh(core_axis_name="c", subcore_axis_name="s",
                                  num_cores=1, num_subcores=NS)
    sc_params = pltpu.CompilerParams(needs_layout_passes=False,
                                     use_tc_tiling_on_sc=False)

    sc = pl.kernel(
        functools.partial(_sc_body, C),
        out_type=jax.ShapeDtypeStruct((S,), jnp.float32),
        mesh=mesh,
        compiler_params=sc_params,
        scratch_types=[
            pltpu.VMEM((ept,), jnp.int32),
            pltpu.VMEM((ept,), jnp.float32),
            pltpu.VMEM((ept,), jnp.float32),
            [pltpu.VMEM((ept // NCK,), jnp.int32) for _ in range(NCK)],
            pltpu.VMEM((npad,), jnp.float32),
            pltpu.VMEM((npad,), jnp.float32),
            pltpu.VMEM((slc,), jnp.float32),
            pltpu.VMEM((spt,), jnp.int32),
            pltpu.VMEM((spt,), jnp.int32),
            pltpu.VMEM((spt,), jnp.float32),
            pltpu.VMEM((L,), jnp.float32),
            pltpu.VMEM_SHARED((npad,), jnp.float32),
            pltpu.SemaphoreType.DMA,
            pltpu.SemaphoreType.DMA,
        ],
    )
    out = sc(edge_index, edge_weight, z2.reshape(N),
             subG_node[:, 0], subG_node[:, 1], c16)
    return out.reshape(S, 1)
